# baseline (speedup 1.0000x reference)
import functools

import jax
import jax.numpy as jnp
from jax import lax
from jax.experimental import pallas as pl
from jax.experimental.pallas import tpu as pltpu
from jax.experimental.pallas import tpu_sc as plsc

N = 10000
E = 320000
D = 128
G = 16

NC = 2
NS = 16
NW = NC * NS

CH = 128
NBUF = 2
IBUF = 4
NPAIR = 160
NIT0 = 108
NIT1 = NPAIR - NIT0
NIT = NPAIR // 2
EPT = NIT * CH
E_PAD = NPAIR * NS * CH
N_SH = 10240
ZCH = N_SH // NS // CH
WB = 632
N_P = WB * NS

B = 1000
NB = N // B


def _sc_agg_body(x_hbm, et_hbm, z_hbm, parts_hbm,
                 idx_r, rows_r, agg_sh, semi, semg):
    cid = lax.axis_index("c")
    sid = lax.axis_index("s")
    base = sid * NPAIR + cid * NIT0
    cnt = jnp.where(cid == 0, NIT0, NIT1)

    def idx_load(ck, j):
        pltpu.async_copy(et_hbm.at[base + ck], idx_r.at[j], semi.at[j])

    def idx_wait(ck, j):
        pltpu.make_async_copy(et_hbm.at[base + ck], idx_r.at[j],
                              semi.at[j]).wait()

    def row_gather(j, b):
        pltpu.async_copy(x_hbm.at[idx_r.at[j, 0]], rows_r.at[b], semg.at[b])

    def row_wait(j, b):
        pltpu.make_async_copy(x_hbm.at[idx_r.at[j, 0]], rows_r.at[b],
                              semg.at[b]).wait()

    pltpu.sync_copy(z_hbm, rows_r.at[0])
    def _zero(k, c):
        pltpu.sync_copy(rows_r.at[0],
                        agg_sh.at[pl.ds(sid * ZCH * CH + k * CH, CH)])
        return c
    lax.fori_loop(0, ZCH, _zero, 0)
    plsc.subcore_barrier()

    for j in range(IBUF):
        idx_load(j, j)
    for b in range(NBUF):
        idx_wait(b, b)
        row_gather(b, b)

    UN = IBUF
    def _round(g, c):
        base = g * UN
        for u in range(UN):
            it = base + u
            b = u % NBUF
            j = u % IBUF
            row_wait(j, b)
            pltpu.sync_copy(rows_r.at[b], agg_sh.at[idx_r.at[j, 1]],
                            add=True)
            nid = it + IBUF

            @pl.when(nid < cnt)
            def _refill():
                idx_load(nid, j)
            ngt = it + NBUF
            j2 = (u + NBUF) % IBUF

            @pl.when(ngt < cnt)
            def _next_gather():
                idx_wait(ngt, j2)
                row_gather(j2, b)
        return c
    lax.fori_loop(0, cnt // UN, _round, 0)
    plsc.subcore_barrier()

    pltpu.sync_copy(agg_sh.at[pl.ds(sid * WB, WB)],
                    parts_hbm.at[cid, pl.ds(sid * WB, WB)])


_sc_agg = pl.kernel(
    _sc_agg_body,
    out_type=jax.ShapeDtypeStruct((NC, N_P, D), jnp.float32),
    mesh=plsc.VectorSubcoreMesh(core_axis_name="c", subcore_axis_name="s"),
    scratch_types=[
        pltpu.VMEM((IBUF, 2, CH), jnp.int32),
        pltpu.VMEM((NBUF, CH, D), jnp.float32),
        pltpu.VMEM_SHARED((N_SH, D), jnp.float32),
        pltpu.SemaphoreType.DMA((IBUF,)),
        pltpu.SemaphoreType.DMA((NBUF,)),
    ],
)


def _tc_body(eps_ref, x_ref, parts_ref, oh_ref, W1_ref, b1_ref, W2_ref,
             b2_ref, vnW_ref, vnb_ref, gamma_ref, beta_ref, out_ref,
             h_s, vn_s, vno_s):
    i = pl.program_id(0)

    @pl.when(i < NB)
    def _phase_a():
        xb = x_ref[...]
        aggb = parts_ref[0] + parts_ref[1]
        h0 = (1.0 + eps_ref[0, 0]) * xb + aggb
        h1 = jnp.maximum(
            jnp.dot(h0, W1_ref[...], preferred_element_type=jnp.float32)
            + b1_ref[...], 0.0)
        h2 = (jnp.dot(h1, W2_ref[...], preferred_element_type=jnp.float32)
              + b2_ref[...])
        hb = xb + h2
        h_s[pl.ds(i * B, B), :] = hb
        contrib = lax.dot_general(oh_ref[...], hb, (((0,), (0,)), ((), ())),
                                  preferred_element_type=jnp.float32)

        @pl.when(i == 0)
        def _init():
            vn_s[...] = contrib

        @pl.when(i > 0)
        def _acc():
            vn_s[...] = vn_s[...] + contrib

    @pl.when(i == NB)
    def _phase_b():
        vn = vn_s[...]
        mean = jnp.mean(vn, axis=0, keepdims=True)
        var = jnp.mean((vn - mean) ** 2, axis=0, keepdims=True)
        vn_n = ((vn - mean) * lax.rsqrt(var + 1e-5) * gamma_ref[...]
                + beta_ref[...])
        vno_s[...] = jnp.maximum(
            jnp.dot(vn_n, vnW_ref[...], preferred_element_type=jnp.float32)
            + vnb_ref[...], 0.0)

    @pl.when(i > NB)
    def _phase_c():
        j = i - NB - 1
        hb = h_s[pl.ds(j * B, B), :]
        out_ref[...] = hb + jnp.dot(oh_ref[...], vno_s[...],
                                    preferred_element_type=jnp.float32)


def _tc_call(eps2, x, parts, onehot, W1, b1, W2, b2, vn_W, vn_b, gamma, beta):
    ia = lambda i: (jnp.minimum(i, NB - 1), 0)
    ic = lambda i: (jnp.clip(i - NB - 1, 0, NB - 1), 0)
    grid = 2 * NB + 1
    return pl.pallas_call(
        _tc_body,
        grid=(grid,),
        in_specs=[
            pl.BlockSpec(memory_space=pltpu.SMEM),
            pl.BlockSpec((B, D), ia),
            pl.BlockSpec((NC, B, D),
                         lambda i: (0, jnp.minimum(i, NB - 1), 0)),
            pl.BlockSpec((B, G),
                         lambda i: (jnp.where(i < NB, i,
                                              jnp.clip(i - NB - 1, 0, NB - 1)),
                                    0)),
            pl.BlockSpec((D, D), lambda i: (0, 0)),
            pl.BlockSpec((1, D), lambda i: (0, 0)),
            pl.BlockSpec((D, D), lambda i: (0, 0)),
            pl.BlockSpec((1, D), lambda i: (0, 0)),
            pl.BlockSpec((D, D), lambda i: (0, 0)),
            pl.BlockSpec((1, D), lambda i: (0, 0)),
            pl.BlockSpec((1, D), lambda i: (0, 0)),
            pl.BlockSpec((1, D), lambda i: (0, 0)),
        ],
        out_specs=pl.BlockSpec((B, D), ic),
        out_shape=jax.ShapeDtypeStruct((N, D), jnp.float32),
        scratch_shapes=[
            pltpu.VMEM((N, D), jnp.float32),
            pltpu.VMEM((G, D), jnp.float32),
            pltpu.VMEM((G, D), jnp.float32),
        ],
        compiler_params=pltpu.CompilerParams(
            dimension_semantics=("arbitrary",)),
    )(eps2, x, parts, onehot, W1, b1, W2, b2, vn_W, vn_b, gamma, beta)


def kernel(x, edge_index, graph_ids, W1, b1, W2, b2, eps, vn_W, vn_b,
           gamma, beta):
    src = edge_index[0]
    dst = edge_index[1]
    pad = E_PAD - E
    srcp = jnp.concatenate([src, jnp.zeros((pad,), jnp.int32)])
    dstp = jnp.concatenate([dst, jnp.full((pad,), N, jnp.int32)])
    et = jnp.stack([srcp.reshape(NS * NPAIR, CH), dstp.reshape(NS * NPAIR, CH)],
                   axis=1)
    zblk = jnp.zeros((CH, D), jnp.float32)

    parts = _sc_agg(x, et, zblk)

    onehot = (graph_ids[:, None]
              == jnp.arange(G, dtype=jnp.int32)[None, :]).astype(jnp.float32)
    return _tc_call(eps.reshape(1, 1), x, parts, onehot, W1,
                    b1.reshape(1, D), W2, b2.reshape(1, D), vn_W,
                    vn_b.reshape(1, D), gamma.reshape(1, D),
                    beta.reshape(1, D))


# speedup vs baseline: 1.0634x; 1.0634x over previous
import functools

import jax
import jax.numpy as jnp
from jax import lax
from jax.experimental import pallas as pl
from jax.experimental.pallas import tpu as pltpu
from jax.experimental.pallas import tpu_sc as plsc

N = 10000
E = 320000
D = 128
G = 16

NC = 2
NS = 16
NW = NC * NS

CH = 128
NBUF = 2
IBUF = 4
NPAIR = 160
NIT0 = 132
NIT1 = NPAIR - NIT0
NIT = NPAIR // 2
EPT = NIT * CH
E_PAD = NPAIR * NS * CH
N_SH = 10240
ZCH = N_SH // NS // CH
WB = 632
N_P = WB * NS

B = 1000
NB = N // B


def _sc_agg_body(x_hbm, et_hbm, z_hbm, parts_hbm,
                 idx_r, rows_r, agg_sh, semi, semg):
    cid = lax.axis_index("c")
    sid = lax.axis_index("s")
    base = sid * NPAIR + cid * NIT0
    cnt = jnp.where(cid == 0, NIT0, NIT1)

    def idx_load(ck, j):
        pltpu.async_copy(et_hbm.at[base + ck], idx_r.at[j], semi.at[j])

    def idx_wait(ck, j):
        pltpu.make_async_copy(et_hbm.at[base + ck], idx_r.at[j],
                              semi.at[j]).wait()

    def row_gather(j, b):
        pltpu.async_copy(x_hbm.at[idx_r.at[j, 0]], rows_r.at[b], semg.at[b])

    def row_wait(j, b):
        pltpu.make_async_copy(x_hbm.at[idx_r.at[j, 0]], rows_r.at[b],
                              semg.at[b]).wait()

    pltpu.sync_copy(z_hbm, rows_r.at[0])
    def _zero(k, c):
        pltpu.sync_copy(rows_r.at[0],
                        agg_sh.at[pl.ds(sid * ZCH * CH + k * CH, CH)])
        return c
    lax.fori_loop(0, ZCH, _zero, 0)
    plsc.subcore_barrier()

    for j in range(IBUF):
        idx_load(j, j)
    for b in range(NBUF):
        idx_wait(b, b)
        row_gather(b, b)

    UN = IBUF
    def _round(g, c):
        base = g * UN
        for u in range(UN):
            it = base + u
            b = u % NBUF
            j = u % IBUF
            row_wait(j, b)
            pltpu.sync_copy(rows_r.at[b], agg_sh.at[idx_r.at[j, 1]],
                            add=True)
            nid = it + IBUF

            @pl.when(nid < cnt)
            def _refill():
                idx_load(nid, j)
            ngt = it + NBUF
            j2 = (u + NBUF) % IBUF

            @pl.when(ngt < cnt)
            def _next_gather():
                idx_wait(ngt, j2)
                row_gather(j2, b)
        return c
    lax.fori_loop(0, cnt // UN, _round, 0)
    plsc.subcore_barrier()

    pltpu.sync_copy(agg_sh.at[pl.ds(sid * WB, WB)],
                    parts_hbm.at[cid, pl.ds(sid * WB, WB)])


_sc_agg = pl.kernel(
    _sc_agg_body,
    out_type=jax.ShapeDtypeStruct((NC, N_P, D), jnp.float32),
    mesh=plsc.VectorSubcoreMesh(core_axis_name="c", subcore_axis_name="s"),
    scratch_types=[
        pltpu.VMEM((IBUF, 2, CH), jnp.int32),
        pltpu.VMEM((NBUF, CH, D), jnp.float32),
        pltpu.VMEM_SHARED((N_SH, D), jnp.float32),
        pltpu.SemaphoreType.DMA((IBUF,)),
        pltpu.SemaphoreType.DMA((NBUF,)),
    ],
)


def _tc_body(eps_ref, x_ref, parts_ref, oh_ref, W1_ref, b1_ref, W2_ref,
             b2_ref, vnW_ref, vnb_ref, gamma_ref, beta_ref, out_ref,
             h_s, vn_s, vno_s):
    i = pl.program_id(0)

    @pl.when(i < NB)
    def _phase_a():
        xb = x_ref[...]
        aggb = parts_ref[0] + parts_ref[1]
        h0 = (1.0 + eps_ref[0, 0]) * xb + aggb
        h1 = jnp.maximum(
            jnp.dot(h0, W1_ref[...], preferred_element_type=jnp.float32)
            + b1_ref[...], 0.0)
        h2 = (jnp.dot(h1, W2_ref[...], preferred_element_type=jnp.float32)
              + b2_ref[...])
        hb = xb + h2
        h_s[pl.ds(i * B, B), :] = hb
        contrib = lax.dot_general(oh_ref[...], hb, (((0,), (0,)), ((), ())),
                                  preferred_element_type=jnp.float32)

        @pl.when(i == 0)
        def _init():
            vn_s[...] = contrib

        @pl.when(i > 0)
        def _acc():
            vn_s[...] = vn_s[...] + contrib

    @pl.when(i == NB)
    def _phase_b():
        vn = vn_s[...]
        mean = jnp.mean(vn, axis=0, keepdims=True)
        var = jnp.mean((vn - mean) ** 2, axis=0, keepdims=True)
        vn_n = ((vn - mean) * lax.rsqrt(var + 1e-5) * gamma_ref[...]
                + beta_ref[...])
        vno_s[...] = jnp.maximum(
            jnp.dot(vn_n, vnW_ref[...], preferred_element_type=jnp.float32)
            + vnb_ref[...], 0.0)

    @pl.when(i > NB)
    def _phase_c():
        j = i - NB - 1
        hb = h_s[pl.ds(j * B, B), :]
        out_ref[...] = hb + jnp.dot(oh_ref[...], vno_s[...],
                                    preferred_element_type=jnp.float32)


def _tc_call(eps2, x, parts, onehot, W1, b1, W2, b2, vn_W, vn_b, gamma, beta):
    ia = lambda i: (jnp.minimum(i, NB - 1), 0)
    ic = lambda i: (jnp.clip(i - NB - 1, 0, NB - 1), 0)
    grid = 2 * NB + 1
    return pl.pallas_call(
        _tc_body,
        grid=(grid,),
        in_specs=[
            pl.BlockSpec(memory_space=pltpu.SMEM),
            pl.BlockSpec((B, D), ia),
            pl.BlockSpec((NC, B, D),
                         lambda i: (0, jnp.minimum(i, NB - 1), 0)),
            pl.BlockSpec((B, G),
                         lambda i: (jnp.where(i < NB, i,
                                              jnp.clip(i - NB - 1, 0, NB - 1)),
                                    0)),
            pl.BlockSpec((D, D), lambda i: (0, 0)),
            pl.BlockSpec((1, D), lambda i: (0, 0)),
            pl.BlockSpec((D, D), lambda i: (0, 0)),
            pl.BlockSpec((1, D), lambda i: (0, 0)),
            pl.BlockSpec((D, D), lambda i: (0, 0)),
            pl.BlockSpec((1, D), lambda i: (0, 0)),
            pl.BlockSpec((1, D), lambda i: (0, 0)),
            pl.BlockSpec((1, D), lambda i: (0, 0)),
        ],
        out_specs=pl.BlockSpec((B, D), ic),
        out_shape=jax.ShapeDtypeStruct((N, D), jnp.float32),
        scratch_shapes=[
            pltpu.VMEM((N, D), jnp.float32),
            pltpu.VMEM((G, D), jnp.float32),
            pltpu.VMEM((G, D), jnp.float32),
        ],
        compiler_params=pltpu.CompilerParams(
            dimension_semantics=("arbitrary",)),
    )(eps2, x, parts, onehot, W1, b1, W2, b2, vn_W, vn_b, gamma, beta)


def kernel(x, edge_index, graph_ids, W1, b1, W2, b2, eps, vn_W, vn_b,
           gamma, beta):
    src = edge_index[0]
    dst = edge_index[1]
    pad = E_PAD - E
    srcp = jnp.concatenate([src, jnp.zeros((pad,), jnp.int32)])
    dstp = jnp.concatenate([dst, jnp.full((pad,), N, jnp.int32)])
    et = jnp.stack([srcp.reshape(NS * NPAIR, CH), dstp.reshape(NS * NPAIR, CH)],
                   axis=1)
    zblk = jnp.zeros((CH, D), jnp.float32)

    parts = _sc_agg(x, et, zblk)

    onehot = (graph_ids[:, None]
              == jnp.arange(G, dtype=jnp.int32)[None, :]).astype(jnp.float32)
    return _tc_call(eps.reshape(1, 1), x, parts, onehot, W1,
                    b1.reshape(1, D), W2, b2.reshape(1, D), vn_W,
                    vn_b.reshape(1, D), gamma.reshape(1, D),
                    beta.reshape(1, D))


# speedup vs baseline: 1.1178x; 1.0512x over previous
import functools

import jax
import jax.numpy as jnp
from jax import lax
from jax.experimental import pallas as pl
from jax.experimental.pallas import tpu as pltpu
from jax.experimental.pallas import tpu_sc as plsc

N = 10000
E = 320000
D = 128
G = 16

NC = 2
NS = 16
NW = NC * NS

CH = 128
NBUF = 2
IBUF = 4
NPAIR = 160
NIT0 = 124
NIT1 = NPAIR - NIT0
NIT = NPAIR // 2
EPT = NIT * CH
E_PAD = NPAIR * NS * CH
N_SH = 10240
ZCH = N_SH // NS // CH
WB = 632
N_P = WB * NS

B = 1000
NB = N // B


def _sc_agg_body(x_hbm, et_hbm, z_hbm, parts_hbm,
                 idx_r, rows_r, agg_sh, semi, semg):
    cid = lax.axis_index("c")
    sid = lax.axis_index("s")
    base = sid * NPAIR + cid * NIT0
    cnt = jnp.where(cid == 0, NIT0, NIT1)

    def idx_load(ck, j):
        pltpu.async_copy(et_hbm.at[base + ck], idx_r.at[j], semi.at[j])

    def idx_wait(ck, j):
        pltpu.make_async_copy(et_hbm.at[base + ck], idx_r.at[j],
                              semi.at[j]).wait()

    def row_gather(j, b):
        pltpu.async_copy(x_hbm.at[idx_r.at[j, 0]], rows_r.at[b], semg.at[b])

    def row_wait(j, b):
        pltpu.make_async_copy(x_hbm.at[idx_r.at[j, 0]], rows_r.at[b],
                              semg.at[b]).wait()

    pltpu.sync_copy(z_hbm, rows_r.at[0])
    def _zero(k, c):
        pltpu.sync_copy(rows_r.at[0],
                        agg_sh.at[pl.ds(sid * ZCH * CH + k * CH, CH)])
        return c
    lax.fori_loop(0, ZCH, _zero, 0)
    plsc.subcore_barrier()

    for j in range(IBUF):
        idx_load(j, j)
    for b in range(NBUF):
        idx_wait(b, b)
        row_gather(b, b)

    UN = IBUF
    def _round(g, c):
        base = g * UN
        for u in range(UN):
            it = base + u
            b = u % NBUF
            j = u % IBUF
            row_wait(j, b)
            pltpu.sync_copy(rows_r.at[b], agg_sh.at[idx_r.at[j, 1]],
                            add=True)
            nid = it + IBUF

            @pl.when(nid < cnt)
            def _refill():
                idx_load(nid, j)
            ngt = it + NBUF
            j2 = (u + NBUF) % IBUF

            @pl.when(ngt < cnt)
            def _next_gather():
                idx_wait(ngt, j2)
                row_gather(j2, b)
        return c
    lax.fori_loop(0, cnt // UN, _round, 0)
    plsc.subcore_barrier()

    pltpu.sync_copy(agg_sh.at[pl.ds(sid * WB, WB)],
                    parts_hbm.at[cid, pl.ds(sid * WB, WB)])


_sc_agg = pl.kernel(
    _sc_agg_body,
    out_type=jax.ShapeDtypeStruct((NC, N_P, D), jnp.float32),
    mesh=plsc.VectorSubcoreMesh(core_axis_name="c", subcore_axis_name="s"),
    scratch_types=[
        pltpu.VMEM((IBUF, 2, CH), jnp.int32),
        pltpu.VMEM((NBUF, CH, D), jnp.float32),
        pltpu.VMEM_SHARED((N_SH, D), jnp.float32),
        pltpu.SemaphoreType.DMA((IBUF,)),
        pltpu.SemaphoreType.DMA((NBUF,)),
    ],
)


def _tc_body(eps_ref, x_ref, parts_ref, oh_ref, W1_ref, b1_ref, W2_ref,
             b2_ref, vnW_ref, vnb_ref, gamma_ref, beta_ref, out_ref,
             h_s, vn_s, vno_s):
    i = pl.program_id(0)

    @pl.when(i < NB)
    def _phase_a():
        xb = x_ref[...]
        aggb = parts_ref[0] + parts_ref[1]
        h0 = (1.0 + eps_ref[0, 0]) * xb + aggb
        h1 = jnp.maximum(
            jnp.dot(h0, W1_ref[...], preferred_element_type=jnp.float32)
            + b1_ref[...], 0.0)
        h2 = (jnp.dot(h1, W2_ref[...], preferred_element_type=jnp.float32)
              + b2_ref[...])
        hb = xb + h2
        h_s[pl.ds(i * B, B), :] = hb
        contrib = lax.dot_general(oh_ref[...], hb, (((0,), (0,)), ((), ())),
                                  preferred_element_type=jnp.float32)

        @pl.when(i == 0)
        def _init():
            vn_s[...] = contrib

        @pl.when(i > 0)
        def _acc():
            vn_s[...] = vn_s[...] + contrib

    @pl.when(i == NB)
    def _phase_b():
        vn = vn_s[...]
        mean = jnp.mean(vn, axis=0, keepdims=True)
        var = jnp.mean((vn - mean) ** 2, axis=0, keepdims=True)
        vn_n = ((vn - mean) * lax.rsqrt(var + 1e-5) * gamma_ref[...]
                + beta_ref[...])
        vno_s[...] = jnp.maximum(
            jnp.dot(vn_n, vnW_ref[...], preferred_element_type=jnp.float32)
            + vnb_ref[...], 0.0)

    @pl.when(i > NB)
    def _phase_c():
        j = i - NB - 1
        hb = h_s[pl.ds(j * B, B), :]
        out_ref[...] = hb + jnp.dot(oh_ref[...], vno_s[...],
                                    preferred_element_type=jnp.float32)


def _tc_call(eps2, x, parts, onehot, W1, b1, W2, b2, vn_W, vn_b, gamma, beta):
    ia = lambda i: (jnp.minimum(i, NB - 1), 0)
    ic = lambda i: (jnp.clip(i - NB - 1, 0, NB - 1), 0)
    grid = 2 * NB + 1
    return pl.pallas_call(
        _tc_body,
        grid=(grid,),
        in_specs=[
            pl.BlockSpec(memory_space=pltpu.SMEM),
            pl.BlockSpec((B, D), ia),
            pl.BlockSpec((NC, B, D),
                         lambda i: (0, jnp.minimum(i, NB - 1), 0)),
            pl.BlockSpec((B, G),
                         lambda i: (jnp.where(i < NB, i,
                                              jnp.clip(i - NB - 1, 0, NB - 1)),
                                    0)),
            pl.BlockSpec((D, D), lambda i: (0, 0)),
            pl.BlockSpec((1, D), lambda i: (0, 0)),
            pl.BlockSpec((D, D), lambda i: (0, 0)),
            pl.BlockSpec((1, D), lambda i: (0, 0)),
            pl.BlockSpec((D, D), lambda i: (0, 0)),
            pl.BlockSpec((1, D), lambda i: (0, 0)),
            pl.BlockSpec((1, D), lambda i: (0, 0)),
            pl.BlockSpec((1, D), lambda i: (0, 0)),
        ],
        out_specs=pl.BlockSpec((B, D), ic),
        out_shape=jax.ShapeDtypeStruct((N, D), jnp.float32),
        scratch_shapes=[
            pltpu.VMEM((N, D), jnp.float32),
            pltpu.VMEM((G, D), jnp.float32),
            pltpu.VMEM((G, D), jnp.float32),
        ],
        compiler_params=pltpu.CompilerParams(
            dimension_semantics=("arbitrary",)),
    )(eps2, x, parts, onehot, W1, b1, W2, b2, vn_W, vn_b, gamma, beta)


def kernel(x, edge_index, graph_ids, W1, b1, W2, b2, eps, vn_W, vn_b,
           gamma, beta):
    src = edge_index[0]
    dst = edge_index[1]
    pad = E_PAD - E
    srcp = jnp.concatenate([src, jnp.zeros((pad,), jnp.int32)])
    dstp = jnp.concatenate([dst, jnp.full((pad,), N, jnp.int32)])
    et = jnp.stack([srcp.reshape(NS * NPAIR, CH), dstp.reshape(NS * NPAIR, CH)],
                   axis=1)
    zblk = jnp.zeros((CH, D), jnp.float32)

    parts = _sc_agg(x, et, zblk)

    onehot = (graph_ids[:, None]
              == jnp.arange(G, dtype=jnp.int32)[None, :]).astype(jnp.float32)
    return _tc_call(eps.reshape(1, 1), x, parts, onehot, W1,
                    b1.reshape(1, D), W2, b2.reshape(1, D), vn_W,
                    vn_b.reshape(1, D), gamma.reshape(1, D),
                    beta.reshape(1, D))


# speedup vs baseline: 1.1179x; 1.0001x over previous
import functools

import jax
import jax.numpy as jnp
from jax import lax
from jax.experimental import pallas as pl
from jax.experimental.pallas import tpu as pltpu
from jax.experimental.pallas import tpu_sc as plsc

N = 10000
E = 320000
D = 128
G = 16

NC = 2
NS = 16
NW = NC * NS

CH = 64
NBUF = 4
IBUF = 8
NPAIR = 320
NIT0 = 248
NIT1 = NPAIR - NIT0
NIT = NPAIR // 2
EPT = NIT * CH
E_PAD = NPAIR * NS * CH
N_SH = 10240
ZCH = N_SH // NS // CH
WB = 632
N_P = WB * NS

B = 1000
NB = N // B


def _sc_agg_body(x_hbm, et_hbm, z_hbm, parts_hbm,
                 idx_r, rows_r, agg_sh, semi, semg):
    cid = lax.axis_index("c")
    sid = lax.axis_index("s")
    base = sid * NPAIR + cid * NIT0
    cnt = jnp.where(cid == 0, NIT0, NIT1)

    def idx_load(ck, j):
        pltpu.async_copy(et_hbm.at[base + ck], idx_r.at[j], semi.at[j])

    def idx_wait(ck, j):
        pltpu.make_async_copy(et_hbm.at[base + ck], idx_r.at[j],
                              semi.at[j]).wait()

    def row_gather(j, b):
        pltpu.async_copy(x_hbm.at[idx_r.at[j, 0]], rows_r.at[b], semg.at[b])

    def row_wait(j, b):
        pltpu.make_async_copy(x_hbm.at[idx_r.at[j, 0]], rows_r.at[b],
                              semg.at[b]).wait()

    pltpu.sync_copy(z_hbm, rows_r.at[0])
    def _zero(k, c):
        pltpu.sync_copy(rows_r.at[0],
                        agg_sh.at[pl.ds(sid * ZCH * CH + k * CH, CH)])
        return c
    lax.fori_loop(0, ZCH, _zero, 0)
    plsc.subcore_barrier()

    for j in range(IBUF):
        idx_load(j, j)
    for b in range(NBUF):
        idx_wait(b, b)
        row_gather(b, b)

    UN = IBUF
    def _round(g, c):
        base = g * UN
        for u in range(UN):
            it = base + u
            b = u % NBUF
            j = u % IBUF
            row_wait(j, b)
            pltpu.sync_copy(rows_r.at[b], agg_sh.at[idx_r.at[j, 1]],
                            add=True)
            nid = it + IBUF

            @pl.when(nid < cnt)
            def _refill():
                idx_load(nid, j)
            ngt = it + NBUF
            j2 = (u + NBUF) % IBUF

            @pl.when(ngt < cnt)
            def _next_gather():
                idx_wait(ngt, j2)
                row_gather(j2, b)
        return c
    lax.fori_loop(0, cnt // UN, _round, 0)
    plsc.subcore_barrier()

    pltpu.sync_copy(agg_sh.at[pl.ds(sid * WB, WB)],
                    parts_hbm.at[cid, pl.ds(sid * WB, WB)])


_sc_agg = pl.kernel(
    _sc_agg_body,
    out_type=jax.ShapeDtypeStruct((NC, N_P, D), jnp.float32),
    mesh=plsc.VectorSubcoreMesh(core_axis_name="c", subcore_axis_name="s"),
    scratch_types=[
        pltpu.VMEM((IBUF, 2, CH), jnp.int32),
        pltpu.VMEM((NBUF, CH, D), jnp.float32),
        pltpu.VMEM_SHARED((N_SH, D), jnp.float32),
        pltpu.SemaphoreType.DMA((IBUF,)),
        pltpu.SemaphoreType.DMA((NBUF,)),
    ],
)


def _tc_body(eps_ref, x_ref, parts_ref, oh_ref, W1_ref, b1_ref, W2_ref,
             b2_ref, vnW_ref, vnb_ref, gamma_ref, beta_ref, out_ref,
             h_s, vn_s, vno_s):
    i = pl.program_id(0)

    @pl.when(i < NB)
    def _phase_a():
        xb = x_ref[...]
        aggb = parts_ref[0] + parts_ref[1]
        h0 = (1.0 + eps_ref[0, 0]) * xb + aggb
        h1 = jnp.maximum(
            jnp.dot(h0, W1_ref[...], preferred_element_type=jnp.float32)
            + b1_ref[...], 0.0)
        h2 = (jnp.dot(h1, W2_ref[...], preferred_element_type=jnp.float32)
              + b2_ref[...])
        hb = xb + h2
        h_s[pl.ds(i * B, B), :] = hb
        contrib = lax.dot_general(oh_ref[...], hb, (((0,), (0,)), ((), ())),
                                  preferred_element_type=jnp.float32)

        @pl.when(i == 0)
        def _init():
            vn_s[...] = contrib

        @pl.when(i > 0)
        def _acc():
            vn_s[...] = vn_s[...] + contrib

    @pl.when(i == NB)
    def _phase_b():
        vn = vn_s[...]
        mean = jnp.mean(vn, axis=0, keepdims=True)
        var = jnp.mean((vn - mean) ** 2, axis=0, keepdims=True)
        vn_n = ((vn - mean) * lax.rsqrt(var + 1e-5) * gamma_ref[...]
                + beta_ref[...])
        vno_s[...] = jnp.maximum(
            jnp.dot(vn_n, vnW_ref[...], preferred_element_type=jnp.float32)
            + vnb_ref[...], 0.0)

    @pl.when(i > NB)
    def _phase_c():
        j = i - NB - 1
        hb = h_s[pl.ds(j * B, B), :]
        out_ref[...] = hb + jnp.dot(oh_ref[...], vno_s[...],
                                    preferred_element_type=jnp.float32)


def _tc_call(eps2, x, parts, onehot, W1, b1, W2, b2, vn_W, vn_b, gamma, beta):
    ia = lambda i: (jnp.minimum(i, NB - 1), 0)
    ic = lambda i: (jnp.clip(i - NB - 1, 0, NB - 1), 0)
    grid = 2 * NB + 1
    return pl.pallas_call(
        _tc_body,
        grid=(grid,),
        in_specs=[
            pl.BlockSpec(memory_space=pltpu.SMEM),
            pl.BlockSpec((B, D), ia),
            pl.BlockSpec((NC, B, D),
                         lambda i: (0, jnp.minimum(i, NB - 1), 0)),
            pl.BlockSpec((B, G),
                         lambda i: (jnp.where(i < NB, i,
                                              jnp.clip(i - NB - 1, 0, NB - 1)),
                                    0)),
            pl.BlockSpec((D, D), lambda i: (0, 0)),
            pl.BlockSpec((1, D), lambda i: (0, 0)),
            pl.BlockSpec((D, D), lambda i: (0, 0)),
            pl.BlockSpec((1, D), lambda i: (0, 0)),
            pl.BlockSpec((D, D), lambda i: (0, 0)),
            pl.BlockSpec((1, D), lambda i: (0, 0)),
            pl.BlockSpec((1, D), lambda i: (0, 0)),
            pl.BlockSpec((1, D), lambda i: (0, 0)),
        ],
        out_specs=pl.BlockSpec((B, D), ic),
        out_shape=jax.ShapeDtypeStruct((N, D), jnp.float32),
        scratch_shapes=[
            pltpu.VMEM((N, D), jnp.float32),
            pltpu.VMEM((G, D), jnp.float32),
            pltpu.VMEM((G, D), jnp.float32),
        ],
        compiler_params=pltpu.CompilerParams(
            dimension_semantics=("arbitrary",)),
    )(eps2, x, parts, onehot, W1, b1, W2, b2, vn_W, vn_b, gamma, beta)


def kernel(x, edge_index, graph_ids, W1, b1, W2, b2, eps, vn_W, vn_b,
           gamma, beta):
    src = edge_index[0]
    dst = edge_index[1]
    pad = E_PAD - E
    srcp = jnp.concatenate([src, jnp.zeros((pad,), jnp.int32)])
    dstp = jnp.concatenate([dst, jnp.full((pad,), N, jnp.int32)])
    et = jnp.stack([srcp.reshape(NS * NPAIR, CH), dstp.reshape(NS * NPAIR, CH)],
                   axis=1)
    zblk = jnp.zeros((CH, D), jnp.float32)

    parts = _sc_agg(x, et, zblk)

    onehot = (graph_ids[:, None]
              == jnp.arange(G, dtype=jnp.int32)[None, :]).astype(jnp.float32)
    return _tc_call(eps.reshape(1, 1), x, parts, onehot, W1,
                    b1.reshape(1, D), W2, b2.reshape(1, D), vn_W,
                    vn_b.reshape(1, D), gamma.reshape(1, D),
                    beta.reshape(1, D))
